# baseline (speedup 1.0000x reference)
import jax
import jax.numpy as jnp
import numpy as np
from jax.experimental import pallas as pl

_LEVELS = np.array([[100, 100], [50, 50], [25, 25], [13, 13]], dtype=np.int64)


def _identity_kernel(x_ref, o_ref):
    o_ref[...] = x_ref[...]


def kernel(value, spatial_shapes, level_start_index, sampling_locations, attention_weights):
    B, L, H, C = value.shape
    _, Q, _, Lv, P, _ = sampling_locations.shape
    out = jnp.zeros((B, Q, H, C), dtype=value.dtype)
    b_idx = jnp.arange(B)[:, None, None, None]
    h_idx = jnp.arange(H)[None, None, :, None]
    for lvl in range(Lv):
        size_l = int(_LEVELS[lvl][0]) * int(_LEVELS[lvl][1])
        Hl = spatial_shapes[lvl, 0]
        Wl = spatial_shapes[lvl, 1]
        s = level_start_index[lvl]
        v = jax.lax.dynamic_slice_in_dim(value, s, size_l, axis=1)
        loc = sampling_locations[:, :, :, lvl]
        x = loc[..., 0] * Wl.astype(value.dtype) - 0.5
        y = loc[..., 1] * Hl.astype(value.dtype) - 0.5
        x0f = jnp.floor(x)
        y0f = jnp.floor(y)
        wx1 = x - x0f
        wx0 = 1.0 - wx1
        wy1 = y - y0f
        wy0 = 1.0 - wy1

        def gather(xi, yi, Wl=Wl, Hl=Hl, v=v):
            xi = xi.astype(jnp.int32)
            yi = yi.astype(jnp.int32)
            Wl_i = Wl.astype(jnp.int32)
            Hl_i = Hl.astype(jnp.int32)
            valid = (xi >= 0) & (xi < Wl_i) & (yi >= 0) & (yi < Hl_i)
            xc = jnp.clip(xi, 0, Wl_i - 1)
            yc = jnp.clip(yi, 0, Hl_i - 1)
            flat = yc * Wl_i + xc
            g = v[b_idx, flat, h_idx]
            return g * valid[..., None].astype(value.dtype)

        samp = (gather(x0f, y0f) * (wx0 * wy0)[..., None]
                + gather(x0f + 1.0, y0f) * (wx1 * wy0)[..., None]
                + gather(x0f, y0f + 1.0) * (wx0 * wy1)[..., None]
                + gather(x0f + 1.0, y0f + 1.0) * (wx1 * wy1)[..., None])
        w = attention_weights[:, :, :, lvl]
        out = out + jnp.sum(samp * w[..., None], axis=3)
    out = out.reshape(B, Q, H * C)
    return pl.pallas_call(
        _identity_kernel,
        out_shape=jax.ShapeDtypeStruct(out.shape, out.dtype),
    )(out)


# speedup vs baseline: 67.3351x; 67.3351x over previous
import functools

import jax
import jax.numpy as jnp
from jax import lax
from jax.experimental import pallas as pl
from jax.experimental.pallas import tpu as pltpu
from jax.experimental.pallas import tpu_sc as plsc

B = 2
L = 13294
Q = 13294
H = 8
C = 32
LV = 4
P = 4
LVL_W = (100, 50, 25, 13)
LVL_H = (100, 50, 25, 13)
LVL_START = (0, 10000, 12500, 13125)

BQ = B * Q
NTILES = 32
ROWS_PER_TILE = 832
BQPAD = NTILES * ROWS_PER_TILE
NB = 4
KCOL = H * LV * P


def _prep_body(locx_ref, locy_ref, aw_ref,
               i0_ref, i1_ref, i2_ref, i3_ref,
               w0_ref, w1_ref, w2_ref, w3_ref):
    rb = locx_ref.shape[0]
    col = lax.broadcasted_iota(jnp.int32, (rb, KCOL), 1)
    h = col >> 4
    lvl = (col >> 2) & 3

    def per_lvl(vals):
        out = jnp.full(col.shape, vals[3], jnp.int32)
        for k in (2, 1, 0):
            out = jnp.where(lvl == k, vals[k], out)
        return out

    wl = per_lvl(LVL_W)
    hl = per_lvl(LVL_H)
    start = per_lvl(LVL_START)

    row = pl.program_id(0) * rb + lax.broadcasted_iota(jnp.int32, (rb, KCOL), 0)
    b = (row >= Q).astype(jnp.int32)

    x = locx_ref[...] * wl.astype(jnp.float32) - 0.5
    y = locy_ref[...] * hl.astype(jnp.float32) - 0.5
    x0f = jnp.floor(x)
    y0f = jnp.floor(y)
    fx1 = x - x0f
    fx0 = 1.0 - fx1
    fy1 = y - y0f
    fy0 = 1.0 - fy1
    x0 = x0f.astype(jnp.int32)
    y0 = y0f.astype(jnp.int32)
    aw = aw_ref[...]

    outs = ((i0_ref, w0_ref, 0, 0), (i1_ref, w1_ref, 0, 1),
            (i2_ref, w2_ref, 1, 0), (i3_ref, w3_ref, 1, 1))
    for i_ref, w_ref, dy, dx in outs:
        xi = x0 + dx
        yi = y0 + dy
        valid = (xi >= 0) & (xi < wl) & (yi >= 0) & (yi < hl)
        xc = jnp.clip(xi, 0, wl - 1)
        yc = jnp.clip(yi, 0, hl - 1)
        flat = start + yc * wl + xc
        i_ref[...] = (b * L + flat) * H + h
        wxy = (fx1 if dx else fx0) * (fy1 if dy else fy0)
        w_ref[...] = wxy * aw * valid.astype(jnp.float32)


def _run_prep(locx, locy, aw):
    nblk = BQPAD // ROWS_PER_TILE
    blk = (ROWS_PER_TILE, KCOL)
    spec = pl.BlockSpec(blk, lambda i: (i, 0))
    shp_i = jax.ShapeDtypeStruct((BQPAD, KCOL), jnp.int32)
    shp_f = jax.ShapeDtypeStruct((BQPAD, KCOL), jnp.float32)
    return pl.pallas_call(
        _prep_body,
        grid=(nblk,),
        in_specs=[spec, spec, spec],
        out_specs=[spec] * 8,
        out_shape=[shp_i] * 4 + [shp_f] * 4,
    )(locx, locy, aw)


def _sc_body(table, i0, i1, i2, i3, w0, w1, w2, w3, out,
             idx_v, wt_v, rows_v, out_v, sem):
    wid = lax.axis_index("s") * 2 + lax.axis_index("c")
    base = wid * ROWS_PER_TILE
    idx_hbm = (i0, i1, i2, i3)
    wt_hbm = (w0, w1, w2, w3)

    def step(it, _):
        row0 = base + it * NB
        for c in range(4):
            pltpu.sync_copy(idx_hbm[c].at[pl.ds(row0, NB)],
                            idx_v.at[pl.ds(c * NB, NB)])
            pltpu.sync_copy(wt_hbm[c].at[pl.ds(row0, NB)],
                            wt_v.at[pl.ds(c * NB, NB)])
        cps = [pltpu.async_copy(table.at[idx_v.at[r]], rows_v.at[r], sem)
               for r in range(4 * NB)]
        for cp in cps:
            cp.wait()

        def per_row(i, _):
            for h in range(H):
                def per_corner(c, accs):
                    acc0, acc1 = accs
                    ridx = c * NB + i
                    wv = wt_v[ridx, pl.ds(h * 16, 16)]
                    for jj in range(16):
                        ws = lax.gather(
                            wv, jnp.full((16, 1), jj, jnp.int32),
                            lax.GatherDimensionNumbers(
                                offset_dims=(), collapsed_slice_dims=(0,),
                                start_index_map=(0,)),
                            (1,), mode=lax.GatherScatterMode.PROMISE_IN_BOUNDS)
                        r0 = rows_v[ridx, h * 16 + jj, pl.ds(0, 16)]
                        r1 = rows_v[ridx, h * 16 + jj, pl.ds(16, 16)]
                        acc0 = acc0 + ws * r0
                        acc1 = acc1 + ws * r1
                    return acc0, acc1

                z = jnp.zeros((16,), jnp.float32)
                acc0, acc1 = lax.fori_loop(0, 4, per_corner, (z, z))
                out_v[i, pl.ds(h * 32, 16)] = acc0
                out_v[i, pl.ds(h * 32 + 16, 16)] = acc1
            return _

        lax.fori_loop(0, NB, per_row, 0)
        pltpu.sync_copy(out_v, out.at[pl.ds(row0, NB)])
        return _

    lax.fori_loop(0, ROWS_PER_TILE // NB, step, 0)


def _run_sc(table, idxs, wts):
    mesh = plsc.VectorSubcoreMesh(core_axis_name="c", subcore_axis_name="s")
    kfn = functools.partial(
        pl.kernel,
        out_type=jax.ShapeDtypeStruct((BQPAD, H * C), jnp.float32),
        mesh=mesh,
        scratch_types=[
            pltpu.VMEM((4 * NB, KCOL), jnp.int32),
            pltpu.VMEM((4 * NB, KCOL), jnp.float32),
            pltpu.VMEM((4 * NB, KCOL, C), jnp.float32),
            pltpu.VMEM((NB, H * C), jnp.float32),
            pltpu.SemaphoreType.DMA,
        ],
        compiler_params=pltpu.CompilerParams(use_tc_tiling_on_sc=False),
    )(_sc_body)
    return kfn(table, *idxs, *wts)


def kernel(value, spatial_shapes, level_start_index, sampling_locations, attention_weights):
    del spatial_shapes, level_start_index
    locx = sampling_locations[..., 0].reshape(BQ, KCOL)
    locy = sampling_locations[..., 1].reshape(BQ, KCOL)
    aw = attention_weights.reshape(BQ, KCOL)
    pad = BQPAD - BQ
    locx = jnp.pad(locx, ((0, pad), (0, 0)))
    locy = jnp.pad(locy, ((0, pad), (0, 0)))
    aw = jnp.pad(aw, ((0, pad), (0, 0)))

    i0, i1, i2, i3, w0, w1, w2, w3 = _run_prep(locx, locy, aw)
    table = value.reshape(B * L * H, C)
    out = _run_sc(table, (i0, i1, i2, i3), (w0, w1, w2, w3))
    return out[:BQ].reshape(B, Q, H * C)


# speedup vs baseline: 117.7590x; 1.7488x over previous
import functools

import jax
import jax.numpy as jnp
from jax import lax
from jax.experimental import pallas as pl
from jax.experimental.pallas import tpu as pltpu
from jax.experimental.pallas import tpu_sc as plsc

B = 2
L = 13294
Q = 13294
H = 8
C = 32
LVL_W = (100, 50, 25, 13)
LVL_H = (100, 50, 25, 13)
LVL_START = (0, 10000, 12500, 13125)

BQ = B * Q
NTILES = 32
ROWS_PER_TILE = 832
BQPAD = NTILES * ROWS_PER_TILE
KCOL = H * 4 * 4
CHUNK = 16
NB = 2
NBATCH = CHUNK // NB
NCHUNK = ROWS_PER_TILE // CHUNK


def _prep_body(locx_ref, locy_ref, aw_ref, ixw_ref):
    rb = locx_ref.shape[0]
    col = lax.broadcasted_iota(jnp.int32, (rb, KCOL), 1)
    h = col >> 4
    lvl = (col >> 2) & 3

    def per_lvl(vals):
        out = jnp.full(col.shape, vals[3], jnp.int32)
        for k in (2, 1, 0):
            out = jnp.where(lvl == k, vals[k], out)
        return out

    wl = per_lvl(LVL_W)
    hl = per_lvl(LVL_H)
    start = per_lvl(LVL_START)

    row = pl.program_id(0) * rb + lax.broadcasted_iota(jnp.int32, (rb, KCOL), 0)
    b = (row >= Q).astype(jnp.int32)

    x = locx_ref[...] * wl.astype(jnp.float32) - 0.5
    y = locy_ref[...] * hl.astype(jnp.float32) - 0.5
    x0f = jnp.floor(x)
    y0f = jnp.floor(y)
    fx1 = x - x0f
    fx0 = 1.0 - fx1
    fy1 = y - y0f
    fy0 = 1.0 - fy1
    x0 = x0f.astype(jnp.int32)
    y0 = y0f.astype(jnp.int32)
    aw = aw_ref[...]

    corners = ((0, 0), (0, 1), (1, 0), (1, 1))
    for c, (dy, dx) in enumerate(corners):
        xi = x0 + dx
        yi = y0 + dy
        valid = (xi >= 0) & (xi < wl) & (yi >= 0) & (yi < hl)
        xc = jnp.clip(xi, 0, wl - 1)
        yc = jnp.clip(yi, 0, hl - 1)
        flat = start + yc * wl + xc
        ixw_ref[:, c, :] = (b * L + flat) * H + h
        wxy = (fx1 if dx else fx0) * (fy1 if dy else fy0)
        w = wxy * aw * valid.astype(jnp.float32)
        ixw_ref[:, 4 + c, :] = lax.bitcast_convert_type(w, jnp.int32)


def _run_prep(locx, locy, aw):
    nblk = BQPAD // ROWS_PER_TILE
    spec = pl.BlockSpec((ROWS_PER_TILE, KCOL), lambda i: (i, 0))
    return pl.pallas_call(
        _prep_body,
        grid=(nblk,),
        in_specs=[spec, spec, spec],
        out_specs=pl.BlockSpec((ROWS_PER_TILE, 8, KCOL), lambda i: (i, 0, 0)),
        out_shape=jax.ShapeDtypeStruct((BQPAD, 8, KCOL), jnp.int32),
    )(locx, locy, aw)


def _reduce_batch(ixw_v, rows_v, out_v, k):
    for i in range(NB):
        def per_h(h, _):
            def per_corner(c, accs):
                wv = plsc.bitcast(
                    ixw_v[k * NB + i, 4 + c, pl.ds(h * 16, 16)], jnp.float32)
                slot = c * NB + i
                new = list(accs)
                for jj in range(16):
                    ws = lax.gather(
                        wv, jnp.full((16, 1), jj, jnp.int32),
                        lax.GatherDimensionNumbers(
                            offset_dims=(), collapsed_slice_dims=(0,),
                            start_index_map=(0,)),
                        (1,), mode=lax.GatherScatterMode.PROMISE_IN_BOUNDS)
                    r0 = rows_v[slot, h * 16 + jj, pl.ds(0, 16)]
                    r1 = rows_v[slot, h * 16 + jj, pl.ds(16, 16)]
                    a = 2 * (jj % 4)
                    new[a] = new[a] + ws * r0
                    new[a + 1] = new[a + 1] + ws * r1
                return tuple(new)

            z = jnp.zeros((16,), jnp.float32)
            accs = lax.fori_loop(0, 4, per_corner, (z,) * 8)
            acc0 = (accs[0] + accs[2]) + (accs[4] + accs[6])
            acc1 = (accs[1] + accs[3]) + (accs[5] + accs[7])
            out_v[k * NB + i, pl.ds(h * 32, 16)] = acc0
            out_v[k * NB + i, pl.ds(h * 32 + 16, 16)] = acc1
            return _

        lax.fori_loop(0, H, per_h, 0)


def _sc_body(table, ixw, out, ixw_a, ixw_b, rows_a, rows_b, out_v,
             sem_ixw, sem_ga, sem_gb):
    wid = lax.axis_index("s") * 2 + lax.axis_index("c")
    base = wid * ROWS_PER_TILE
    rows_bufs = (rows_a, rows_b)
    gat_sems = (sem_ga, sem_gb)

    def load_ixw(chunk_idx, dst):
        return pltpu.async_copy(
            ixw.at[pl.ds(base + chunk_idx * CHUNK, CHUNK)], dst, sem_ixw)

    def issue_gathers(ixw_v, k):
        cps = []
        for c in range(4):
            for i in range(NB):
                cps.append(pltpu.async_copy(
                    table.at[ixw_v.at[k * NB + i, c]],
                    rows_bufs[k % 2].at[c * NB + i],
                    gat_sems[k % 2]))
        return cps

    load_ixw(0, ixw_a)

    def half(u, cc, ixw_v, ixw_other):
        pltpu.make_async_copy(ixw.at[pl.ds(base, CHUNK)], ixw_v, sem_ixw).wait()
        if ixw_other is not None:
            nxt = cc + 1

            @pl.when(nxt < NCHUNK)
            def _():
                load_ixw(nxt, ixw_other)

        pending = issue_gathers(ixw_v, 0)
        for k in range(NBATCH):
            if k + 1 < NBATCH:
                nxt_pending = issue_gathers(ixw_v, k + 1)
            for cp in pending:
                cp.wait()
            _reduce_batch(ixw_v, rows_bufs[k % 2], out_v, k)
            if k + 1 < NBATCH:
                pending = nxt_pending
        pltpu.sync_copy(out_v, out.at[pl.ds(base + cc * CHUNK, CHUNK)])

    def pair(u, _):
        half(u, 2 * u, ixw_a, ixw_b)
        half(u, 2 * u + 1, ixw_b, ixw_a)
        return _

    lax.fori_loop(0, NCHUNK // 2, pair, 0)


def _run_sc(table, ixw):
    mesh = plsc.VectorSubcoreMesh(core_axis_name="c", subcore_axis_name="s")
    kfn = functools.partial(
        pl.kernel,
        out_type=jax.ShapeDtypeStruct((BQPAD, H * C), jnp.float32),
        mesh=mesh,
        scratch_types=[
            pltpu.VMEM((CHUNK, 8, KCOL), jnp.int32),
            pltpu.VMEM((CHUNK, 8, KCOL), jnp.int32),
            pltpu.VMEM((4 * NB, KCOL, C), jnp.float32),
            pltpu.VMEM((4 * NB, KCOL, C), jnp.float32),
            pltpu.VMEM((CHUNK, H * C), jnp.float32),
            pltpu.SemaphoreType.DMA,
            pltpu.SemaphoreType.DMA,
            pltpu.SemaphoreType.DMA,
        ],
        compiler_params=pltpu.CompilerParams(
            use_tc_tiling_on_sc=False, needs_layout_passes=False),
    )(_sc_body)
    return kfn(table, ixw)


def kernel(value, spatial_shapes, level_start_index, sampling_locations, attention_weights):
    del spatial_shapes, level_start_index
    locx = sampling_locations[..., 0].reshape(BQ, KCOL)
    locy = sampling_locations[..., 1].reshape(BQ, KCOL)
    aw = attention_weights.reshape(BQ, KCOL)
    pad = BQPAD - BQ
    locx = jnp.pad(locx, ((0, pad), (0, 0)))
    locy = jnp.pad(locy, ((0, pad), (0, 0)))
    aw = jnp.pad(aw, ((0, pad), (0, 0)))

    ixw = _run_prep(locx, locy, aw)
    table = value.reshape(B * L * H, C)
    out = _run_sc(table, ixw)
    return out[:BQ].reshape(B, Q, H * C)
